# baseline (speedup 1.0000x reference)
import functools
import math

import jax
import jax.numpy as jnp
import numpy as np
from jax import lax
from jax.experimental import pallas as pl
from jax.experimental.pallas import tpu as pltpu
from jax.experimental.pallas import tpu_sc as plsc

D = 64
MAX_NORM = 2.0
SQRT_D = math.sqrt(D)
POLY_DEG = 3


def _pe_coeff_table() -> np.ndarray:
    d = np.arange(D, dtype=np.float64)
    c = d / 10000.0 ** (2.0 * d / D)
    tg = np.linspace(0.0, 1.0, 1024)
    ctab = np.empty((POLY_DEG + 1, D), dtype=np.float64)
    for dd in range(D):
        f = np.sin(c[dd] * tg) if dd % 2 == 0 else np.cos(c[dd] * tg)
        ctab[:, dd] = np.polyfit(tg, f, POLY_DEG)
    return ctab.astype(np.float32)


_CTAB = _pe_coeff_table()

_INFO = plsc.get_sparse_core_info()
_NC, _NS = _INFO.num_cores, _INFO.num_subcores
_NW = _NC * _NS
_B, _L = 1024, 200
_N_TOK = _B * _L
_TPW = _N_TOK // _NW
_CH = 128
_N_CHUNK = _TPW // _CH
_UNROLL = 8


def _body(mz_h, int_h, tab_h, ctab_h, out_h, idx_v, t_v, rows_v, out_t_v,
          ctab_v, gsem, wsem):
    wid = lax.axis_index("s") * _NC + lax.axis_index("c")
    base0 = wid * _TPW

    pltpu.sync_copy(ctab_h, ctab_v)
    pltpu.sync_copy(mz_h.at[pl.ds(base0, _TPW)], idx_v)
    pltpu.sync_copy(int_h.at[pl.ds(base0, _TPW)], t_v)
    C = [[ctab_v[m, pl.ds(k * 16, 16)] for k in range(4)]
         for m in range(POLY_DEG + 1)]
    iota = lax.iota(jnp.int32, 16)

    def fire_gather(ci, buf):
        boff = pl.multiple_of(buf * _CH, _CH)
        pltpu.async_copy(
            tab_h.at[idx_v.at[pl.ds(ci * _CH, _CH)]],
            rows_v.at[pl.ds(boff, _CH)],
            gsem,
        )

    fire_gather(0, 0)

    def chunk_body(ci, carry):
        cur = lax.rem(ci, 2)
        coff = pl.multiple_of(cur * _CH, _CH)
        tbase = ci * _CH
        gtok = base0 + tbase
        l_pos = lax.shift_right_logical(gtok, 10)
        b0 = pl.multiple_of(lax.rem(gtok, _B), _CH)
        rowvec = [iota + (cur * D + 16 * k) for k in range(4)]

        @pl.when(ci < _N_CHUNK - 1)
        def _prefetch():
            fire_gather(ci + 1, lax.rem(ci + 1, 2))

        @pl.when(ci >= 2)
        def _drain_write():
            pltpu.make_async_copy(
                out_h.at[pl.ds(0, D), pl.ds(0, _CH)],
                out_t_v.at[pl.ds(cur * D, D), pl.ds(0, _CH)], wsem).wait()

        pltpu.make_async_copy(
            tab_h.at[pl.ds(0, _CH)],
            rows_v.at[pl.ds(coff, _CH)], gsem).wait()

        def tok_body(g, carry2):
            s8s = []
            for u in range(_UNROLL):
                tok = coff + g * _UNROLL + u
                acc = None
                for k in range(4):
                    rk = rows_v[tok, pl.ds(k * 16, 16)]
                    sq = rk * rk
                    acc = sq if acc is None else acc + sq
                ns = jnp.sum(acc)
                i = lax.bitcast_convert_type(ns, jnp.int32)
                i = jnp.int32(0x5F3759DF) - lax.shift_right_logical(i, 1)
                y = lax.bitcast_convert_type(i, jnp.float32)
                y = y * (1.5 - (ns * 0.5) * y * y)
                s8s.append(jnp.minimum(SQRT_D * MAX_NORM * y, SQRT_D))
            for u in range(_UNROLL):
                tokc = g * _UNROLL + u
                tok = coff + tokc
                s8v = jnp.broadcast_to(s8s[u], (16,))
                colvec = jnp.broadcast_to(tokc, (16,)).astype(jnp.int32)
                tsplat = plsc.load_gather(
                    t_v,
                    [jnp.broadcast_to(tbase + tokc, (16,)).astype(jnp.int32)])
                for k in range(4):
                    pe = C[0][k]
                    for m in range(1, POLY_DEG + 1):
                        pe = pe * tsplat + C[m][k]
                    rk = rows_v[tok, pl.ds(k * 16, 16)]
                    plsc.store_scatter(out_t_v, [rowvec[k], colvec],
                                       rk * s8v + pe)
            return carry2

        lax.fori_loop(0, _CH // _UNROLL, tok_body, 0, unroll=False)
        for dk8 in range(0, D, 8):
            pltpu.async_copy(
                out_t_v.at[pl.ds(pl.multiple_of(cur * D + dk8, 8), 8),
                           pl.ds(0, _CH)],
                out_h.at[pl.ds(l_pos * D + dk8, 8), pl.ds(b0, _CH)],
                wsem,
            )
        return carry

    lax.fori_loop(0, _N_CHUNK, chunk_body, 0, unroll=False)
    for buf in range(2):
        pltpu.make_async_copy(
            out_h.at[pl.ds(0, D), pl.ds(0, _CH)],
            out_t_v.at[pl.ds(buf * D, D), pl.ds(0, _CH)], wsem).wait()


def kernel(mz_batch, int_batch, table):
    B, L = mz_batch.shape
    mz_flat = mz_batch.astype(jnp.int32).T.reshape(_N_TOK)
    int_flat = int_batch.T.reshape(_N_TOK)
    table_pad = jnp.pad(table, ((0, 0), (0, D)))
    ctab = jnp.asarray(_CTAB)

    mesh = plsc.VectorSubcoreMesh(core_axis_name="c", subcore_axis_name="s")
    run = functools.partial(
        pl.kernel,
        mesh=mesh,
        out_type=jax.ShapeDtypeStruct((_L * D, _B), jnp.float32),
        scratch_types=[
            pltpu.VMEM((_TPW,), jnp.int32),
            pltpu.VMEM((_TPW,), jnp.float32),
            pltpu.VMEM((2 * _CH, 2 * D), jnp.float32),
            pltpu.VMEM((2 * D, _CH), jnp.float32),
            pltpu.VMEM((POLY_DEG + 1, D), jnp.float32),
            pltpu.SemaphoreType.DMA,
            pltpu.SemaphoreType.DMA,
        ],
        compiler_params=pltpu.CompilerParams(
            needs_layout_passes=False, use_tc_tiling_on_sc=True),
    )(_body)
    out = run(mz_flat, int_flat, table_pad, ctab)
    return out.reshape(_L, D, _B).transpose(2, 0, 1)


# speedup vs baseline: 1.3175x; 1.3175x over previous
import functools
import math

import jax
import jax.numpy as jnp
import numpy as np
from jax import lax
from jax.experimental import pallas as pl
from jax.experimental.pallas import tpu as pltpu
from jax.experimental.pallas import tpu_sc as plsc

D = 64
MAX_NORM = 2.0
SQRT_D = math.sqrt(D)
POLY_DEG = 3


def _pe_coeff_table() -> np.ndarray:
    d = np.arange(D, dtype=np.float64)
    c = d / 10000.0 ** (2.0 * d / D)
    tg = np.linspace(0.0, 1.0, 1024)
    ctab = np.empty((POLY_DEG + 1, D), dtype=np.float64)
    for dd in range(D):
        f = np.sin(c[dd] * tg) if dd % 2 == 0 else np.cos(c[dd] * tg)
        ctab[:, dd] = np.polyfit(tg, f, POLY_DEG)
    return ctab.astype(np.float32)


_CTAB = _pe_coeff_table()

_INFO = plsc.get_sparse_core_info()
_NC, _NS = _INFO.num_cores, _INFO.num_subcores
_NW = _NC * _NS
_N_TOK = 1024 * 200
_TPW = _N_TOK // _NW
_CH = 128
_N_CHUNK = _TPW // _CH
_UNROLL = 8


def _body(mz_h, int_h, tab_h, ctab_h, out_h, idx_v, t_v, rows_v, out_v,
          ctab_v, gsem, wsem):
    wid = lax.axis_index("s") * _NC + lax.axis_index("c")
    base0 = wid * _TPW

    pltpu.sync_copy(ctab_h, ctab_v)
    pltpu.sync_copy(mz_h.at[pl.ds(base0, _TPW)], idx_v)
    pltpu.sync_copy(int_h.at[pl.ds(base0, _TPW)], t_v)
    C = [[ctab_v[m, pl.ds(k * 16, 16)] for k in range(4)]
         for m in range(POLY_DEG + 1)]

    def fire_gather(ci, buf):
        boff = pl.multiple_of(buf * _CH, _CH)
        pltpu.async_copy(
            tab_h.at[idx_v.at[pl.ds(ci * _CH, _CH)]],
            rows_v.at[pl.ds(boff, _CH)],
            gsem,
        )

    fire_gather(0, 0)

    def chunk_body(ci, carry):
        cur = lax.rem(ci, 2)
        coff = pl.multiple_of(cur * _CH, _CH)
        tbase = ci * _CH

        @pl.when(ci < _N_CHUNK - 1)
        def _prefetch():
            fire_gather(ci + 1, lax.rem(ci + 1, 2))

        @pl.when(ci >= 2)
        def _drain_write():
            pltpu.make_async_copy(
                out_h.at[pl.ds(0, _CH)],
                out_v.at[pl.ds(coff, _CH)], wsem).wait()

        pltpu.make_async_copy(
            tab_h.at[pl.ds(0, _CH)],
            rows_v.at[pl.ds(coff, _CH)], gsem).wait()

        @plsc.parallel_loop(0, _CH, step=1, unroll=_UNROLL)
        def _tok(tokc):
            tok = coff + tokc
            r = [rows_v[tok, pl.ds(k * 16, 16)] for k in range(4)]
            acc = r[0] * r[0]
            for k in range(1, 4):
                acc = acc + r[k] * r[k]
            ns = jnp.sum(acc)
            i = lax.bitcast_convert_type(ns, jnp.int32)
            i = jnp.int32(0x5F3759DF) - lax.shift_right_logical(i, 1)
            y = lax.bitcast_convert_type(i, jnp.float32)
            y = y * (1.5 - (ns * 0.5) * y * y)
            s8 = jnp.minimum(SQRT_D * MAX_NORM * y, SQRT_D)
            s8v = jnp.broadcast_to(s8, (16,))
            tsplat = plsc.load_gather(
                t_v, [jnp.broadcast_to(tbase + tokc, (16,)).astype(jnp.int32)])
            for k in range(4):
                pe = C[0][k]
                for m in range(1, POLY_DEG + 1):
                    pe = pe * tsplat + C[m][k]
                out_v[tok, pl.ds(k * 16, 16)] = r[k] * s8v + pe
        pltpu.async_copy(
            out_v.at[pl.ds(coff, _CH)],
            out_h.at[pl.ds(base0 + tbase, _CH)], wsem)
        return carry

    lax.fori_loop(0, _N_CHUNK, chunk_body, 0, unroll=False)
    for buf in range(2):
        pltpu.make_async_copy(
            out_h.at[pl.ds(0, _CH)],
            out_v.at[pl.ds(buf * _CH, _CH)], wsem).wait()


def kernel(mz_batch, int_batch, table):
    B, L = mz_batch.shape
    mz_flat = mz_batch.astype(jnp.int32).reshape(_N_TOK)
    int_flat = int_batch.reshape(_N_TOK)
    table_pad = jnp.pad(table, ((0, 0), (0, D)))
    ctab = jnp.asarray(_CTAB)

    mesh = plsc.VectorSubcoreMesh(core_axis_name="c", subcore_axis_name="s")
    run = functools.partial(
        pl.kernel,
        mesh=mesh,
        out_type=jax.ShapeDtypeStruct((_N_TOK, D), jnp.float32),
        scratch_types=[
            pltpu.VMEM((_TPW,), jnp.int32),
            pltpu.VMEM((_TPW,), jnp.float32),
            pltpu.VMEM((2 * _CH, 2 * D), jnp.float32),
            pltpu.VMEM((2 * _CH, D), jnp.float32),
            pltpu.VMEM((POLY_DEG + 1, D), jnp.float32),
            pltpu.SemaphoreType.DMA,
            pltpu.SemaphoreType.DMA,
        ],
        compiler_params=pltpu.CompilerParams(
            needs_layout_passes=False, use_tc_tiling_on_sc=True),
    )(_body)
    out = run(mz_flat, int_flat, table_pad, ctab)
    return out.reshape(B, L, D)


# speedup vs baseline: 1.3413x; 1.0181x over previous
import functools
import math

import jax
import jax.numpy as jnp
import numpy as np
from jax import lax
from jax.experimental import pallas as pl
from jax.experimental.pallas import tpu as pltpu
from jax.experimental.pallas import tpu_sc as plsc

D = 64
MAX_NORM = 2.0
SQRT_D = math.sqrt(D)
POLY_DEG = 3


def _pe_coeff_table() -> np.ndarray:
    d = np.arange(D, dtype=np.float64)
    c = d / 10000.0 ** (2.0 * d / D)
    tg = np.linspace(0.0, 1.0, 1024)
    ctab = np.empty((POLY_DEG + 1, D), dtype=np.float64)
    for dd in range(D):
        f = np.sin(c[dd] * tg) if dd % 2 == 0 else np.cos(c[dd] * tg)
        ctab[:, dd] = np.polyfit(tg, f, POLY_DEG)
    return ctab.astype(np.float32)


_CTAB = _pe_coeff_table()

_INFO = plsc.get_sparse_core_info()
_NC, _NS = _INFO.num_cores, _INFO.num_subcores
_NW = _NC * _NS
_N_TOK = 1024 * 200
_TPW = _N_TOK // _NW
_CH = 128
_N_CHUNK = _TPW // _CH
_UNROLL = 4


def _body(mz_h, int_h, tab_h, ctab_h, out_h, idx_v, t_v, rows_v, out_v,
          ctab_v, gsem, wsem):
    wid = lax.axis_index("s") * _NC + lax.axis_index("c")
    base0 = wid * _TPW

    pltpu.sync_copy(ctab_h, ctab_v)
    pltpu.sync_copy(mz_h.at[pl.ds(base0, _TPW)], idx_v)
    pltpu.sync_copy(int_h.at[pl.ds(base0, _TPW)], t_v)
    C = [[ctab_v[m, pl.ds(k * 16, 16)] for k in range(4)]
         for m in range(POLY_DEG + 1)]

    def fire_gather(ci, buf):
        boff = pl.multiple_of(buf * _CH, _CH)
        pltpu.async_copy(
            tab_h.at[idx_v.at[pl.ds(ci * _CH, _CH)]],
            rows_v.at[pl.ds(boff, _CH)],
            gsem,
        )

    fire_gather(0, 0)

    def chunk_body(ci, carry):
        cur = lax.rem(ci, 2)
        coff = pl.multiple_of(cur * _CH, _CH)
        tbase = ci * _CH

        @pl.when(ci < _N_CHUNK - 1)
        def _prefetch():
            fire_gather(ci + 1, lax.rem(ci + 1, 2))

        @pl.when(ci >= 2)
        def _drain_write():
            pltpu.make_async_copy(
                out_h.at[pl.ds(0, _CH)],
                out_v.at[pl.ds(coff, _CH)], wsem).wait()

        pltpu.make_async_copy(
            tab_h.at[pl.ds(0, _CH)],
            rows_v.at[pl.ds(coff, _CH)], gsem).wait()

        @plsc.parallel_loop(0, _CH, step=1, unroll=_UNROLL)
        def _tok(tokc):
            tok = coff + tokc
            r = [rows_v[tok, pl.ds(k * 16, 16)] for k in range(4)]
            acc = r[0] * r[0]
            for k in range(1, 4):
                acc = acc + r[k] * r[k]
            ns = jnp.sum(acc)
            i = lax.bitcast_convert_type(ns, jnp.int32)
            i = jnp.int32(0x5F3759DF) - lax.shift_right_logical(i, 1)
            y = lax.bitcast_convert_type(i, jnp.float32)
            y = y * (1.5 - (ns * 0.5) * y * y)
            s8 = jnp.minimum(SQRT_D * MAX_NORM * y, SQRT_D)
            s8v = jnp.broadcast_to(s8, (16,))
            tsplat = plsc.load_gather(
                t_v, [jnp.broadcast_to(tbase + tokc, (16,)).astype(jnp.int32)])
            for k in range(4):
                pe = C[0][k]
                for m in range(1, POLY_DEG + 1):
                    pe = pe * tsplat + C[m][k]
                out_v[tok, pl.ds(k * 16, 16)] = r[k] * s8v + pe
        pltpu.async_copy(
            out_v.at[pl.ds(coff, _CH)],
            out_h.at[pl.ds(base0 + tbase, _CH)], wsem)
        return carry

    lax.fori_loop(0, _N_CHUNK, chunk_body, 0, unroll=False)
    for buf in range(2):
        pltpu.make_async_copy(
            out_h.at[pl.ds(0, _CH)],
            out_v.at[pl.ds(buf * _CH, _CH)], wsem).wait()


def kernel(mz_batch, int_batch, table):
    B, L = mz_batch.shape
    mz_flat = mz_batch.astype(jnp.int32).reshape(_N_TOK)
    int_flat = int_batch.reshape(_N_TOK)
    table_pad = jnp.pad(table, ((0, 0), (0, D)))
    ctab = jnp.asarray(_CTAB)

    mesh = plsc.VectorSubcoreMesh(core_axis_name="c", subcore_axis_name="s")
    run = functools.partial(
        pl.kernel,
        mesh=mesh,
        out_type=jax.ShapeDtypeStruct((_N_TOK, D), jnp.float32),
        scratch_types=[
            pltpu.VMEM((_TPW,), jnp.int32),
            pltpu.VMEM((_TPW,), jnp.float32),
            pltpu.VMEM((2 * _CH, 2 * D), jnp.float32),
            pltpu.VMEM((2 * _CH, D), jnp.float32),
            pltpu.VMEM((POLY_DEG + 1, D), jnp.float32),
            pltpu.SemaphoreType.DMA,
            pltpu.SemaphoreType.DMA,
        ],
        compiler_params=pltpu.CompilerParams(
            needs_layout_passes=False, use_tc_tiling_on_sc=True),
    )(_body)
    out = run(mz_flat, int_flat, table_pad, ctab)
    return out.reshape(B, L, D)


# speedup vs baseline: 1.3415x; 1.0002x over previous
import functools
import math

import jax
import jax.numpy as jnp
import numpy as np
from jax import lax
from jax.experimental import pallas as pl
from jax.experimental.pallas import tpu as pltpu
from jax.experimental.pallas import tpu_sc as plsc

D = 64
MAX_NORM = 2.0
SQRT_D = math.sqrt(D)
POLY_DEG = 3


def _pe_coeff_table() -> np.ndarray:
    d = np.arange(D, dtype=np.float64)
    c = d / 10000.0 ** (2.0 * d / D)
    tg = np.linspace(0.0, 1.0, 1024)
    ctab = np.empty((POLY_DEG + 1, D), dtype=np.float64)
    for dd in range(D):
        f = np.sin(c[dd] * tg) if dd % 2 == 0 else np.cos(c[dd] * tg)
        ctab[:, dd] = np.polyfit(tg, f, POLY_DEG)
    return ctab.astype(np.float32)


_CTAB = _pe_coeff_table()

_INFO = plsc.get_sparse_core_info()
_NC, _NS = _INFO.num_cores, _INFO.num_subcores
_NW = _NC * _NS
_N_TOK = 1024 * 200
_TPW = _N_TOK // _NW
_CH = 128
_N_CHUNK = _TPW // _CH
_UNROLL = 2


def _body(mz_h, int_h, tab_h, ctab_h, out_h, idx_v, t_v, rows_v, out_v,
          ctab_v, gsem, wsem):
    wid = lax.axis_index("s") * _NC + lax.axis_index("c")
    base0 = wid * _TPW

    pltpu.sync_copy(ctab_h, ctab_v)
    pltpu.sync_copy(mz_h.at[pl.ds(base0, _TPW)], idx_v)
    pltpu.sync_copy(int_h.at[pl.ds(base0, _TPW)], t_v)
    C = [[ctab_v[m, pl.ds(k * 16, 16)] for k in range(4)]
         for m in range(POLY_DEG + 1)]

    def fire_gather(ci, buf):
        boff = pl.multiple_of(buf * _CH, _CH)
        pltpu.async_copy(
            tab_h.at[idx_v.at[pl.ds(ci * _CH, _CH)]],
            rows_v.at[pl.ds(boff, _CH)],
            gsem,
        )

    fire_gather(0, 0)

    def chunk_body(ci, carry):
        cur = lax.rem(ci, 2)
        coff = pl.multiple_of(cur * _CH, _CH)
        tbase = ci * _CH

        @pl.when(ci < _N_CHUNK - 1)
        def _prefetch():
            fire_gather(ci + 1, lax.rem(ci + 1, 2))

        @pl.when(ci >= 2)
        def _drain_write():
            pltpu.make_async_copy(
                out_h.at[pl.ds(0, _CH)],
                out_v.at[pl.ds(coff, _CH)], wsem).wait()

        pltpu.make_async_copy(
            tab_h.at[pl.ds(0, _CH)],
            rows_v.at[pl.ds(coff, _CH)], gsem).wait()

        @plsc.parallel_loop(0, _CH, step=1, unroll=_UNROLL)
        def _tok(tokc):
            tok = coff + tokc
            r = [rows_v[tok, pl.ds(k * 16, 16)] for k in range(4)]
            acc = r[0] * r[0]
            for k in range(1, 4):
                acc = acc + r[k] * r[k]
            ns = jnp.sum(acc)
            i = lax.bitcast_convert_type(ns, jnp.int32)
            i = jnp.int32(0x5F3759DF) - lax.shift_right_logical(i, 1)
            y = lax.bitcast_convert_type(i, jnp.float32)
            y = y * (1.5 - (ns * 0.5) * y * y)
            s8 = jnp.minimum(SQRT_D * MAX_NORM * y, SQRT_D)
            s8v = jnp.broadcast_to(s8, (16,))
            tsplat = plsc.load_gather(
                t_v, [jnp.broadcast_to(tbase + tokc, (16,)).astype(jnp.int32)])
            for k in range(4):
                pe = C[0][k]
                for m in range(1, POLY_DEG + 1):
                    pe = pe * tsplat + C[m][k]
                out_v[tok, pl.ds(k * 16, 16)] = r[k] * s8v + pe
        pltpu.async_copy(
            out_v.at[pl.ds(coff, _CH)],
            out_h.at[pl.ds(base0 + tbase, _CH)], wsem)
        return carry

    lax.fori_loop(0, _N_CHUNK, chunk_body, 0, unroll=False)
    for buf in range(2):
        pltpu.make_async_copy(
            out_h.at[pl.ds(0, _CH)],
            out_v.at[pl.ds(buf * _CH, _CH)], wsem).wait()


def kernel(mz_batch, int_batch, table):
    B, L = mz_batch.shape
    mz_flat = mz_batch.astype(jnp.int32).reshape(_N_TOK)
    int_flat = int_batch.reshape(_N_TOK)
    table_pad = jnp.pad(table, ((0, 0), (0, D)))
    ctab = jnp.asarray(_CTAB)

    mesh = plsc.VectorSubcoreMesh(core_axis_name="c", subcore_axis_name="s")
    run = functools.partial(
        pl.kernel,
        mesh=mesh,
        out_type=jax.ShapeDtypeStruct((_N_TOK, D), jnp.float32),
        scratch_types=[
            pltpu.VMEM((_TPW,), jnp.int32),
            pltpu.VMEM((_TPW,), jnp.float32),
            pltpu.VMEM((2 * _CH, 2 * D), jnp.float32),
            pltpu.VMEM((2 * _CH, D), jnp.float32),
            pltpu.VMEM((POLY_DEG + 1, D), jnp.float32),
            pltpu.SemaphoreType.DMA,
            pltpu.SemaphoreType.DMA,
        ],
        compiler_params=pltpu.CompilerParams(
            needs_layout_passes=False, use_tc_tiling_on_sc=True),
    )(_body)
    out = run(mz_flat, int_flat, table_pad, ctab)
    return out.reshape(B, L, D)
